# Initial kernel scaffold; baseline (speedup 1.0000x reference)
import jax
import jax.numpy as jnp
from jax.experimental import pallas as pl


def kernel(sent, table):
    raise NotImplementedError("write your pallas kernel here")


# speedup vs baseline: 1.3066x; 1.3066x over previous
import functools

import jax
import jax.numpy as jnp
from jax import lax
from jax.experimental import pallas as pl
from jax.experimental.pallas import tpu as pltpu
from jax.experimental.pallas import tpu_sc as plsc

_CHUNK = 128
_NC = 2
_NS = 16
_NW = _NC * _NS


def _make_gather(n_rows, dim):
    n_chunks = n_rows // _CHUNK
    nch = n_chunks // _NW
    mesh = plsc.VectorSubcoreMesh(core_axis_name="c", subcore_axis_name="s")

    @functools.partial(
        pl.kernel,
        mesh=mesh,
        out_type=jax.ShapeDtypeStruct((n_rows, dim), jnp.float32),
        scratch_types=[
            pltpu.VMEM((nch, _CHUNK), jnp.int32),
            pltpu.VMEM((_CHUNK, dim), jnp.float32),
            pltpu.SemaphoreType.DMA,
        ],
        compiler_params=pltpu.CompilerParams(use_tc_tiling_on_sc=False),
    )
    def k(table_hbm, idx_hbm, out_hbm, idx_v, buf, sem):
        wid = lax.axis_index("s") * _NC + lax.axis_index("c")
        ch0 = wid * nch
        out0 = ch0 * _CHUNK
        pltpu.sync_copy(idx_hbm.at[pl.ds(ch0, nch)], idx_v)

        def body(j, carry):
            pltpu.async_copy(table_hbm.at[idx_v.at[j]], buf, sem).wait()
            pltpu.sync_copy(buf, out_hbm.at[pl.ds(out0 + j * _CHUNK, _CHUNK)])
            return carry

        lax.fori_loop(0, nch, body, 0)

    return k


def kernel(sent, table):
    b, h = sent.shape
    n = b * h
    idx2d = sent.reshape(n // _CHUNK, _CHUNK)
    out = _make_gather(n, table.shape[1])(table, idx2d)
    return out.reshape(b, h, table.shape[1])


# speedup vs baseline: 1.4931x; 1.1427x over previous
import functools

import jax
import jax.numpy as jnp
from jax import lax
from jax.experimental import pallas as pl
from jax.experimental.pallas import tpu as pltpu
from jax.experimental.pallas import tpu_sc as plsc

_CHUNK = 128
_K = 8
_NC = 2
_NS = 16
_NW = _NC * _NS


def _make_gather(n_rows, dim):
    n_chunks = n_rows // _CHUNK
    nch = n_chunks // _NW
    n_ph = nch // _K
    half = _K * _CHUNK
    mesh = plsc.VectorSubcoreMesh(core_axis_name="c", subcore_axis_name="s")

    @functools.partial(
        pl.kernel,
        mesh=mesh,
        out_type=jax.ShapeDtypeStruct((n_rows, dim), jnp.float32),
        scratch_types=[
            pltpu.VMEM((nch, _CHUNK), jnp.int32),
            pltpu.VMEM((2 * half, dim), jnp.float32),
            pltpu.SemaphoreType.DMA((2,)),
            pltpu.SemaphoreType.DMA((2,)),
        ],
        compiler_params=pltpu.CompilerParams(use_tc_tiling_on_sc=False),
    )
    def k(table_hbm, idx_hbm, out_hbm, idx_v, buf, gsem, osem):
        wid = lax.axis_index("s") * _NC + lax.axis_index("c")
        ch0 = wid * nch
        out0 = ch0 * _CHUNK
        pltpu.sync_copy(idx_hbm.at[pl.ds(ch0, nch)], idx_v)

        def fire_gathers(ph, par):
            for t in range(_K):
                pltpu.async_copy(
                    table_hbm.at[idx_v.at[ph * _K + t]],
                    buf.at[pl.ds(par * half + t * _CHUNK, _CHUNK)],
                    gsem.at[par],
                )

        def drain_gathers(par):
            pltpu.make_async_copy(
                table_hbm.at[pl.ds(0, half)],
                buf.at[pl.ds(par * half, half)],
                gsem.at[par],
            ).wait()

        def fire_ocopy(ph, par):
            pltpu.async_copy(
                buf.at[pl.ds(par * half, half)],
                out_hbm.at[pl.ds(out0 + ph * half, half)],
                osem.at[par],
            )

        def wait_ocopy(par):
            pltpu.make_async_copy(
                buf.at[pl.ds(par * half, half)],
                out_hbm.at[pl.ds(out0, half)],
                osem.at[par],
            ).wait()

        fire_gathers(0, 0)

        def body(p, carry):
            par = lax.rem(p, 2)
            prev = 1 - par
            drain_gathers(prev)
            fire_ocopy(p - 1, prev)

            @pl.when(p >= 2)
            def _():
                wait_ocopy(par)

            @pl.when(p < n_ph)
            def _():
                fire_gathers(p, par)

            return carry

        lax.fori_loop(1, n_ph + 1, body, 0)
        wait_ocopy((n_ph - 1) % 2)

    return k


def kernel(sent, table):
    b, h = sent.shape
    n = b * h
    idx2d = sent.reshape(n // _CHUNK, _CHUNK)
    out = _make_gather(n, table.shape[1])(table, idx2d)
    return out.reshape(b, h, table.shape[1])
